# baseline (speedup 1.0000x reference)
import functools
import jax
import jax.numpy as jnp
from jax import lax
from jax.experimental import pallas as pl
from jax.experimental.pallas import tpu as pltpu
from jax.experimental.pallas import tpu_sc as plsc

N = 10000
NP = 10240
E = 320000
DIN = 128
H1 = 128
H2 = 96
OUT = 20
OUTP = 32
GPAD = 128
GRID = 200
P = 66
NCELL = GRID * GRID
HALF = NCELL // 2
ACC3 = HALF + 16

NC = 2
NS = 16
NW = NC * NS
EPW = E // NW
EK = 80
ENCH = EPW // EK
PPW = NP // NW
PB = 64


_MESH = plsc.VectorSubcoreMesh(core_axis_name="c", subcore_axis_name="s")
_BISECT = True


def _zero_buf(zb, c):
    z16 = jnp.zeros((16,), jnp.float32)
    for i in range(16):
        for j in range(c // 16):
            zb[i, pl.ds(j * 16, 16)] = z16


def _make_segsum(c):
    rpt = NP // NS

    @functools.partial(
        pl.kernel,
        out_type=jax.ShapeDtypeStruct((NC, NP, c), jnp.float32),
        mesh=_MESH,
        scratch_types=[
            pltpu.VMEM((EK,), jnp.int32),
            pltpu.VMEM((EK,), jnp.int32),
            pltpu.VMEM((EK, c), jnp.float32),
            pltpu.VMEM((16, c), jnp.float32),
            pltpu.VMEM_SHARED((NP, c), jnp.float32),
            pltpu.SemaphoreType.DMA,
        ],
    )
    def k(feats, srci, dsti, out, src_v, dst_v, rows_v, zb, acc, sem):
        cid = lax.axis_index("c")
        sid = lax.axis_index("s")
        wid = sid * NC + cid

        _zero_buf(zb, c)

        def zacc(i, carry):
            pltpu.sync_copy(zb, acc.at[pl.ds(sid * rpt + i * 16, 16)])
            return carry
        lax.fori_loop(0, rpt // 16, zacc, 0)
        plsc.subcore_barrier()

        base0 = wid * EPW

        def body(kk, carry):
            base = kk * EK
            pltpu.sync_copy(srci.at[pl.ds(base, EK)], src_v)
            pltpu.sync_copy(dsti.at[pl.ds(base, EK)], dst_v)
            pltpu.async_copy(feats.at[src_v], rows_v, sem).wait()
            pltpu.sync_copy(rows_v, acc.at[dst_v], add=True)
            return carry

        @pl.when(wid == 0)
        def _single_worker_probe():
            lax.fori_loop(0, ENCH * NW, body, 0)
        plsc.subcore_barrier()

        pltpu.sync_copy(acc.at[pl.ds(sid * rpt, rpt)],
                        out.at[cid, pl.ds(sid * rpt, rpt)])

    return k


_segsum128 = _make_segsum(DIN)


def _tc1(p, W0):
    def body(p_ref, w0_ref, h_ref):
        agg = p_ref[0] + p_ref[1]
        h_ref[...] = jnp.maximum(
            jnp.dot(agg, w0_ref[...], preferred_element_type=jnp.float32), 0.0)

    return pl.pallas_call(
        body,
        out_shape=jax.ShapeDtypeStruct((NP, H1), jnp.float32),
    )(p, W0)


def _tc_h1(q, W1):
    def body(q_ref, w1_ref, h_ref):
        agg = q_ref[0] + q_ref[1]
        h_ref[...] = jnp.maximum(
            jnp.dot(agg, w1_ref[...], preferred_element_type=jnp.float32), 0.0)

    return pl.pallas_call(
        body,
        out_shape=jax.ShapeDtypeStruct((NP, H2), jnp.float32),
    )(q, W1)


@functools.partial(
    pl.kernel,
    out_type=(
        jax.ShapeDtypeStruct((NCELL, H2), jnp.float32),
        jax.ShapeDtypeStruct((NP,), jnp.int32),
    ),
    mesh=_MESH,
    scratch_types=[
        pltpu.VMEM((PB,), jnp.float32),
        pltpu.VMEM((PB,), jnp.float32),
        pltpu.VMEM((PB, H2), jnp.float32),
        pltpu.VMEM((PB,), jnp.int32),
        pltpu.VMEM((PB,), jnp.int32),
        pltpu.VMEM((16, H2), jnp.float32),
        pltpu.VMEM_SHARED((ACC3, H2), jnp.float32),
        pltpu.SemaphoreType.DMA,
    ],
)
def _bev_sc(h1, cxi, cyi, bev_out, pidx_out,
            cx_v, cy_v, h1_v, idx_v, pidx_v, zb, acc, sem):
    cid = lax.axis_index("c")
    sid = lax.axis_index("s")
    wid = sid * NC + cid

    _zero_buf(zb, H2)
    zpt = ACC3 // NS

    def zacc(i, carry):
        pltpu.sync_copy(zb, acc.at[pl.ds(sid * zpt + i * 16, 16)])
        return carry
    lax.fori_loop(0, zpt // 16, zacc, 0)
    rem = zpt % 16
    if rem:
        pltpu.sync_copy(zb.at[pl.ds(0, rem)],
                        acc.at[pl.ds(sid * zpt + (zpt // 16) * 16, rem)])
    plsc.subcore_barrier()

    c100 = jnp.float32(100.0)
    c50 = jnp.float32(50.0)
    chalf = jnp.float32(0.5)
    lane = lax.iota(jnp.int32, 16)

    for u in range(PPW // PB):
        base = wid * PPW + u * PB
        pltpu.sync_copy(cxi.at[pl.ds(base, PB)], cx_v)
        pltpu.sync_copy(cyi.at[pl.ds(base, PB)], cy_v)
        pltpu.sync_copy(h1.at[pl.ds(base, PB)], h1_v)

        for t in range(PB // 16):
            s = t * 16
            xs = cx_v[pl.ds(s, 16)]
            ys = cy_v[pl.ds(s, 16)]
            fx = ((xs * c100 - c50) + c50) / chalf
            px = jnp.clip(fx.astype(jnp.int32), 0, GRID - 1)
            fy = ((ys * c100 - c50) + c50) / chalf
            py = jnp.clip(GRID - fy.astype(jnp.int32) - 1, 0, GRID - 1)
            qcell = py * GRID + px
            rowid = base + s + lane
            qc = qcell - cid * HALF
            own = (rowid < N) & (qc >= 0) & (qc < HALF)
            idx_v[pl.ds(s, 16)] = jnp.where(own, qc, HALF + (qcell & 15))
            third = jnp.float32(1.0 / 3.0)
            ppy = jnp.clip(
                ((py + 1).astype(jnp.float32) * third).astype(jnp.int32), 0, P - 1)
            ppx = jnp.clip(
                ((px + 1).astype(jnp.float32) * third).astype(jnp.int32), 0, P - 1)
            pidx_v[pl.ds(s, 16)] = ppy * P + ppx

        pltpu.sync_copy(pidx_v, pidx_out.at[pl.ds(base, PB)])
        pltpu.sync_copy(h1_v, acc.at[idx_v], add=True)

    plsc.subcore_barrier()

    opt = 1248
    pltpu.sync_copy(acc.at[pl.ds(sid * opt, opt)],
                    bev_out.at[pl.ds(cid * HALF + sid * opt, opt)])
    tail = HALF - NS * opt

    @pl.when(sid < tail // 8)
    def _copy_tail():
        pltpu.sync_copy(acc.at[pl.ds(NS * opt + sid * 8, 8)],
                        bev_out.at[pl.ds(cid * HALF + NS * opt + sid * 8, 8)])


def _tc2(bev3, h1, Wfp, bfp):
    def body(bev_ref, h1_ref, wf_ref, bf_ref, pw_ref, hw_ref):
        bev = bev_ref[...]
        ninf = jnp.float32(-jnp.inf)

        a = jnp.max(bev[0:198].reshape(P, 3, GRID, H2), axis=1)
        sh = bev[2:200].reshape(P, 3, GRID, H2)
        b = jnp.concatenate(
            [jnp.full((1, GRID, H2), ninf, jnp.float32), sh[:P - 1, 0]], axis=0)
        o1 = jnp.maximum(jnp.maximum(a, b), sh[:, 1])

        cols = []
        for j in range(P):
            lo = max(3 * j - 1, 0)
            hi = 3 * j + 4
            cols.append(jnp.max(o1[:, lo:hi, :], axis=1, keepdims=True))
        pooled = jnp.concatenate(cols, axis=1).reshape(P * P, H2)

        pw_ref[...] = jnp.dot(pooled, wf_ref[...],
                              preferred_element_type=jnp.float32)
        hw_ref[...] = jnp.dot(h1_ref[...], wf_ref[:, :OUTP],
                              preferred_element_type=jnp.float32) + bf_ref[...]

    return pl.pallas_call(
        body,
        out_shape=(
            jax.ShapeDtypeStruct((P * P, GPAD), jnp.float32),
            jax.ShapeDtypeStruct((NP, OUTP), jnp.float32),
        ),
    )(bev3, h1, Wfp, bfp)


@functools.partial(
    pl.kernel,
    out_type=jax.ShapeDtypeStruct((NP, OUTP), jnp.float32),
    mesh=_MESH,
    scratch_types=[
        pltpu.VMEM((PB,), jnp.int32),
        pltpu.VMEM((PB, OUTP), jnp.float32),
        pltpu.VMEM((PB, GPAD), jnp.float32),
        pltpu.SemaphoreType.DMA,
    ],
)
def _final_sc(h1w, pooledw, pidx, out, pidx_v, hw_v, g_v, sem):
    cid = lax.axis_index("c")
    sid = lax.axis_index("s")
    wid = sid * NC + cid

    for u in range(PPW // PB):
        base = wid * PPW + u * PB
        pltpu.sync_copy(pidx.at[pl.ds(base, PB)], pidx_v)
        pltpu.sync_copy(h1w.at[pl.ds(base, PB)], hw_v)
        pltpu.async_copy(pooledw.at[pidx_v], g_v, sem).wait()

        def row(i, carry):
            for j in range(OUTP // 16):
                hw_v[i, pl.ds(j * 16, 16)] = (hw_v[i, pl.ds(j * 16, 16)]
                                              + g_v[i, pl.ds(j * 16, 16)])
            return carry
        lax.fori_loop(0, PB, row, 0)

        pltpu.sync_copy(hw_v, out.at[pl.ds(base, PB)])


def kernel(x, edge_index, coords, W0, W1, Wf, bf):
    src = edge_index[0]
    dst = edge_index[1]
    cx = jnp.pad(coords[:, 0], (0, NP - N))
    cy = jnp.pad(coords[:, 1], (0, NP - N))

    p = _segsum128(x, src, dst)
    if _BISECT:
        h0 = jnp.maximum((p[0] + p[1]) @ W0, 0.0)
        q1j = jax.ops.segment_sum(h0[:N][src], dst, num_segments=N)
        q1j = jnp.pad(q1j, ((0, NP - N), (0, 0)))
        h1 = jnp.maximum(q1j @ W1, 0.0)
    else:
        h0 = _tc1(p, W0)
        q = _segsum128(h0, src, dst)
        h1 = _tc_h1(q, W1)
    if _BISECT:
        t0 = cx * jnp.float32(100.0) - jnp.float32(50.0)
        fx = (t0 + jnp.float32(50.0)) / jnp.float32(0.5)
        px = jnp.clip(fx.astype(jnp.int32), 0, GRID - 1)
        t1 = cy * jnp.float32(100.0) - jnp.float32(50.0)
        fy = (t1 + jnp.float32(50.0)) / jnp.float32(0.5)
        py = jnp.clip(GRID - fy.astype(jnp.int32) - 1, 0, GRID - 1)
        qcell = py * GRID + px
        validm = jnp.arange(NP) < N
        bev = jnp.zeros((NCELL, H2)).at[
            jnp.where(validm, qcell, NCELL)].add(h1, mode='drop')
        ppy = jnp.clip((py + 1) // 3, 0, P - 1)
        ppx = jnp.clip((px + 1) // 3, 0, P - 1)
        pidx = ppy * P + ppx
    else:
        bev, pidx = _bev_sc(h1, cx, cy)

    Wfp = jnp.pad(Wf, ((0, 0), (0, GPAD - OUT)))
    bfp = jnp.pad(bf, (0, OUTP - OUT)).reshape(1, OUTP)
    pooledW, h1W = _tc2(bev.reshape(GRID, GRID, H2), h1, Wfp, bfp)
    if _BISECT:
        outp = h1W + pooledW[pidx, :OUTP]
    else:
        outp = _final_sc(h1W, pooledW, pidx)
    return outp[:N, :OUT]


# speedup vs baseline: 1.2322x; 1.2322x over previous
import functools
import jax
import jax.numpy as jnp
from jax import lax
from jax.experimental import pallas as pl
from jax.experimental.pallas import tpu as pltpu
from jax.experimental.pallas import tpu_sc as plsc

N = 10000
NP = 10240
E = 320000
DIN = 128
H1 = 128
H2 = 96
OUT = 20
OUTP = 32
GPAD = 128
GRID = 200
P = 66
NCELL = GRID * GRID
HALF = NCELL // 2
ACC3 = HALF + 16

NC = 2
NS = 16
NW = NC * NS
EPW = E // NW
EK = 80
ENCH = EPW // EK
PPW = NP // NW
PB = 64


_MESH = plsc.VectorSubcoreMesh(core_axis_name="c", subcore_axis_name="s")
_BISECT = True


def _zero_buf(zb, c):
    z16 = jnp.zeros((16,), jnp.float32)
    for i in range(16):
        for j in range(c // 16):
            zb[i, pl.ds(j * 16, 16)] = z16


def _make_segsum(c):
    rpt = NP // NS

    @functools.partial(
        pl.kernel,
        out_type=jax.ShapeDtypeStruct((NC, NP, c), jnp.float32),
        mesh=_MESH,
        scratch_types=[
            pltpu.VMEM((EK,), jnp.int32),
            pltpu.VMEM((EK,), jnp.int32),
            pltpu.VMEM((EK, c), jnp.float32),
            pltpu.VMEM((16, c), jnp.float32),
            pltpu.VMEM_SHARED((NP, c), jnp.float32),
            pltpu.SemaphoreType.DMA,
        ],
    )
    def k(feats, srci, dsti, out, src_v, dst_v, rows_v, zb, acc, sem):
        cid = lax.axis_index("c")
        sid = lax.axis_index("s")
        wid = sid * NC + cid

        _zero_buf(zb, c)

        def zacc(i, carry):
            pltpu.sync_copy(zb, acc.at[pl.ds(sid * rpt + i * 16, 16)])
            return carry
        lax.fori_loop(0, rpt // 16, zacc, 0)
        plsc.subcore_barrier()

        base0 = wid * EPW

        def body(kk, carry):
            base = base0 + kk * EK
            pltpu.sync_copy(srci.at[pl.ds(base, EK)], src_v)
            pltpu.sync_copy(dsti.at[pl.ds(base, EK)], dst_v)
            pltpu.async_copy(feats.at[src_v], rows_v, sem).wait()
            pltpu.sync_copy(rows_v, acc.at[dst_v], add=True)
            return carry
        lax.fori_loop(0, ENCH, body, 0)
        plsc.subcore_barrier()

        pltpu.sync_copy(acc.at[pl.ds(sid * rpt, rpt)],
                        out.at[cid, pl.ds(sid * rpt, rpt)])

    return k


_segsum128 = _make_segsum(DIN)


def _tc1(p, W0):
    def body(p_ref, w0_ref, h_ref):
        agg = p_ref[0] + p_ref[1]
        h_ref[...] = jnp.maximum(
            jnp.dot(agg, w0_ref[...], preferred_element_type=jnp.float32), 0.0)

    return pl.pallas_call(
        body,
        out_shape=jax.ShapeDtypeStruct((NP, H1), jnp.float32),
    )(p, W0)


def _tc_h1(q, W1):
    def body(q_ref, w1_ref, h_ref):
        agg = q_ref[0] + q_ref[1]
        h_ref[...] = jnp.maximum(
            jnp.dot(agg, w1_ref[...], preferred_element_type=jnp.float32), 0.0)

    return pl.pallas_call(
        body,
        out_shape=jax.ShapeDtypeStruct((NP, H2), jnp.float32),
    )(q, W1)


@functools.partial(
    pl.kernel,
    out_type=(
        jax.ShapeDtypeStruct((NCELL, H2), jnp.float32),
        jax.ShapeDtypeStruct((NP,), jnp.int32),
    ),
    mesh=_MESH,
    scratch_types=[
        pltpu.VMEM((PB,), jnp.float32),
        pltpu.VMEM((PB,), jnp.float32),
        pltpu.VMEM((PB, H2), jnp.float32),
        pltpu.VMEM((PB,), jnp.int32),
        pltpu.VMEM((PB,), jnp.int32),
        pltpu.VMEM((16, H2), jnp.float32),
        pltpu.VMEM_SHARED((ACC3, H2), jnp.float32),
        pltpu.SemaphoreType.DMA,
    ],
)
def _bev_sc(h1, cxi, cyi, bev_out, pidx_out,
            cx_v, cy_v, h1_v, idx_v, pidx_v, zb, acc, sem):
    cid = lax.axis_index("c")
    sid = lax.axis_index("s")
    wid = sid * NC + cid

    _zero_buf(zb, H2)
    zpt = ACC3 // NS

    def zacc(i, carry):
        pltpu.sync_copy(zb, acc.at[pl.ds(sid * zpt + i * 16, 16)])
        return carry
    lax.fori_loop(0, zpt // 16, zacc, 0)
    rem = zpt % 16
    if rem:
        pltpu.sync_copy(zb.at[pl.ds(0, rem)],
                        acc.at[pl.ds(sid * zpt + (zpt // 16) * 16, rem)])
    plsc.subcore_barrier()

    c100 = jnp.float32(100.0)
    c50 = jnp.float32(50.0)
    chalf = jnp.float32(0.5)
    lane = lax.iota(jnp.int32, 16)

    for u in range(PPW // PB):
        base = wid * PPW + u * PB
        pltpu.sync_copy(cxi.at[pl.ds(base, PB)], cx_v)
        pltpu.sync_copy(cyi.at[pl.ds(base, PB)], cy_v)
        pltpu.sync_copy(h1.at[pl.ds(base, PB)], h1_v)

        for t in range(PB // 16):
            s = t * 16
            xs = cx_v[pl.ds(s, 16)]
            ys = cy_v[pl.ds(s, 16)]
            fx = ((xs * c100 - c50) + c50) / chalf
            px = jnp.clip(fx.astype(jnp.int32), 0, GRID - 1)
            fy = ((ys * c100 - c50) + c50) / chalf
            py = jnp.clip(GRID - fy.astype(jnp.int32) - 1, 0, GRID - 1)
            qcell = py * GRID + px
            rowid = base + s + lane
            qc = qcell - cid * HALF
            own = (rowid < N) & (qc >= 0) & (qc < HALF)
            idx_v[pl.ds(s, 16)] = jnp.where(own, qc, HALF + (qcell & 15))
            third = jnp.float32(1.0 / 3.0)
            ppy = jnp.clip(
                ((py + 1).astype(jnp.float32) * third).astype(jnp.int32), 0, P - 1)
            ppx = jnp.clip(
                ((px + 1).astype(jnp.float32) * third).astype(jnp.int32), 0, P - 1)
            pidx_v[pl.ds(s, 16)] = ppy * P + ppx

        pltpu.sync_copy(pidx_v, pidx_out.at[pl.ds(base, PB)])
        pltpu.sync_copy(h1_v, acc.at[idx_v], add=True)

    plsc.subcore_barrier()

    opt = 1248
    pltpu.sync_copy(acc.at[pl.ds(sid * opt, opt)],
                    bev_out.at[pl.ds(cid * HALF + sid * opt, opt)])
    tail = HALF - NS * opt

    @pl.when(sid < tail // 8)
    def _copy_tail():
        pltpu.sync_copy(acc.at[pl.ds(NS * opt + sid * 8, 8)],
                        bev_out.at[pl.ds(cid * HALF + NS * opt + sid * 8, 8)])


def _tc2(bev3, h1, Wfp, bfp):
    def body(bev_ref, h1_ref, wf_ref, bf_ref, pw_ref, hw_ref):
        bev = bev_ref[...]
        ninf = jnp.float32(-jnp.inf)

        a = jnp.max(bev[0:198].reshape(P, 3, GRID, H2), axis=1)
        sh = bev[2:200].reshape(P, 3, GRID, H2)
        b = jnp.concatenate(
            [jnp.full((1, GRID, H2), ninf, jnp.float32), sh[:P - 1, 0]], axis=0)
        o1 = jnp.maximum(jnp.maximum(a, b), sh[:, 1])

        cols = []
        for j in range(P):
            lo = max(3 * j - 1, 0)
            hi = 3 * j + 4
            cols.append(jnp.max(o1[:, lo:hi, :], axis=1, keepdims=True))
        pooled = jnp.concatenate(cols, axis=1).reshape(P * P, H2)

        pw_ref[...] = jnp.dot(pooled, wf_ref[...],
                              preferred_element_type=jnp.float32)
        hw_ref[...] = jnp.dot(h1_ref[...], wf_ref[:, :OUTP],
                              preferred_element_type=jnp.float32) + bf_ref[...]

    return pl.pallas_call(
        body,
        out_shape=(
            jax.ShapeDtypeStruct((P * P, GPAD), jnp.float32),
            jax.ShapeDtypeStruct((NP, OUTP), jnp.float32),
        ),
    )(bev3, h1, Wfp, bfp)


@functools.partial(
    pl.kernel,
    out_type=jax.ShapeDtypeStruct((NP, OUTP), jnp.float32),
    mesh=_MESH,
    scratch_types=[
        pltpu.VMEM((PB,), jnp.int32),
        pltpu.VMEM((PB, OUTP), jnp.float32),
        pltpu.VMEM((PB, GPAD), jnp.float32),
        pltpu.SemaphoreType.DMA,
    ],
)
def _final_sc(h1w, pooledw, pidx, out, pidx_v, hw_v, g_v, sem):
    cid = lax.axis_index("c")
    sid = lax.axis_index("s")
    wid = sid * NC + cid

    for u in range(PPW // PB):
        base = wid * PPW + u * PB
        pltpu.sync_copy(pidx.at[pl.ds(base, PB)], pidx_v)
        pltpu.sync_copy(h1w.at[pl.ds(base, PB)], hw_v)
        pltpu.async_copy(pooledw.at[pidx_v], g_v, sem).wait()

        def row(i, carry):
            for j in range(OUTP // 16):
                hw_v[i, pl.ds(j * 16, 16)] = (hw_v[i, pl.ds(j * 16, 16)]
                                              + g_v[i, pl.ds(j * 16, 16)])
            return carry
        lax.fori_loop(0, PB, row, 0)

        pltpu.sync_copy(hw_v, out.at[pl.ds(base, PB)])


def kernel(x, edge_index, coords, W0, W1, Wf, bf):
    src = edge_index[0]
    dst = edge_index[1]
    cx = jnp.pad(coords[:, 0], (0, NP - N))
    cy = jnp.pad(coords[:, 1], (0, NP - N))

    p = _segsum128(x, src, dst)
    h0 = _tc1(p, W0)
    q = _segsum128(h0, src, dst)
    h1 = _tc_h1(q, W1)
    if _BISECT:
        t0 = cx * jnp.float32(100.0) - jnp.float32(50.0)
        fx = (t0 + jnp.float32(50.0)) / jnp.float32(0.5)
        px = jnp.clip(fx.astype(jnp.int32), 0, GRID - 1)
        t1 = cy * jnp.float32(100.0) - jnp.float32(50.0)
        fy = (t1 + jnp.float32(50.0)) / jnp.float32(0.5)
        py = jnp.clip(GRID - fy.astype(jnp.int32) - 1, 0, GRID - 1)
        qcell = py * GRID + px
        validm = jnp.arange(NP) < N
        bev = jnp.zeros((NCELL, H2)).at[
            jnp.where(validm, qcell, NCELL)].add(h1, mode='drop')
        ppy = jnp.clip((py + 1) // 3, 0, P - 1)
        ppx = jnp.clip((px + 1) // 3, 0, P - 1)
        pidx = ppy * P + ppx
    else:
        bev, pidx = _bev_sc(h1, cx, cy)

    Wfp = jnp.pad(Wf, ((0, 0), (0, GPAD - OUT)))
    bfp = jnp.pad(bf, (0, OUTP - OUT)).reshape(1, OUTP)
    pooledW, h1W = _tc2(bev.reshape(GRID, GRID, H2), h1, Wfp, bfp)
    if _BISECT:
        outp = h1W + pooledW[pidx, :OUTP]
    else:
        outp = _final_sc(h1W, pooledW, pidx)
    return outp[:N, :OUT]


# speedup vs baseline: 62.2154x; 50.4912x over previous
import functools
import jax
import jax.numpy as jnp
from jax import lax
from jax.experimental import pallas as pl
from jax.experimental.pallas import tpu as pltpu
from jax.experimental.pallas import tpu_sc as plsc

N = 10000
NP = 10240
E = 320000
DIN = 128
H1 = 128
H2 = 96
OUT = 20
OUTP = 32
GPAD = 128
GRID = 200
P = 66

NC = 2
NS = 16
NW = NC * NS
EPW = E // NW
EK = 80
ENCH = EPW // EK
PPW = NP // NW
PB = 64

_MESH = plsc.VectorSubcoreMesh(core_axis_name="c", subcore_axis_name="s")


def _zero_buf(zb):
    z16 = jnp.zeros((16,), jnp.float32)
    for i in range(16):
        for j in range(8):
            zb[i, pl.ds(j * 16, 16)] = z16


@functools.partial(
    pl.kernel,
    out_type=jax.ShapeDtypeStruct((NC, NP, DIN), jnp.float32),
    mesh=_MESH,
    scratch_types=[
        pltpu.VMEM((EK,), jnp.int32),
        pltpu.VMEM((EK,), jnp.int32),
        pltpu.VMEM((EK, DIN), jnp.float32),
        pltpu.VMEM((16, DIN), jnp.float32),
        pltpu.VMEM_SHARED((NP, DIN), jnp.float32),
        pltpu.SemaphoreType.DMA,
    ],
)
def _segsum(feats, srci, dsti, out, src_v, dst_v, rows_v, zb, acc, sem):
    cid = lax.axis_index("c")
    sid = lax.axis_index("s")
    wid = sid * NC + cid
    rpt = NP // NS

    _zero_buf(zb)

    def zacc(i, carry):
        pltpu.sync_copy(zb, acc.at[pl.ds(sid * rpt + i * 16, 16)])
        return carry
    lax.fori_loop(0, rpt // 16, zacc, 0)
    plsc.subcore_barrier()

    base0 = wid * EPW

    def body(kk, carry):
        base = base0 + kk * EK
        pltpu.sync_copy(srci.at[pl.ds(base, EK)], src_v)
        pltpu.sync_copy(dsti.at[pl.ds(base, EK)], dst_v)
        pltpu.async_copy(feats.at[src_v], rows_v, sem).wait()
        pltpu.sync_copy(rows_v, acc.at[dst_v], add=True)
        return carry
    lax.fori_loop(0, ENCH, body, 0)
    plsc.subcore_barrier()

    pltpu.sync_copy(acc.at[pl.ds(sid * rpt, rpt)],
                    out.at[cid, pl.ds(sid * rpt, rpt)])


def _tc1(p, W0):
    def body(p_ref, w0_ref, h_ref):
        agg = p_ref[0] + p_ref[1]
        h_ref[...] = jnp.maximum(
            jnp.dot(agg, w0_ref[...], preferred_element_type=jnp.float32), 0.0)

    return pl.pallas_call(
        body,
        out_shape=jax.ShapeDtypeStruct((NP, H1), jnp.float32),
    )(p, W0)


def _tc_h1(q, W1):
    def body(q_ref, w1_ref, h_ref):
        agg = q_ref[0] + q_ref[1]
        h_ref[...] = jnp.maximum(
            jnp.dot(agg, w1_ref[...], preferred_element_type=jnp.float32), 0.0)

    return pl.pallas_call(
        body,
        out_shape=jax.ShapeDtypeStruct((NP, H2), jnp.float32),
    )(q, W1)


def _tc2(bev3, h1, Wfp, bfp):
    def body(bev_ref, h1_ref, wf_ref, bf_ref, pw_ref, hw_ref):
        bev = bev_ref[...]
        ninf = jnp.float32(-jnp.inf)

        a = jnp.max(bev[0:198].reshape(P, 3, GRID, H2), axis=1)
        sh = bev[2:200].reshape(P, 3, GRID, H2)
        b = jnp.concatenate(
            [jnp.full((1, GRID, H2), ninf, jnp.float32), sh[:P - 1, 0]], axis=0)
        o1 = jnp.maximum(jnp.maximum(a, b), sh[:, 1])

        cols = []
        for j in range(P):
            lo = max(3 * j - 1, 0)
            hi = 3 * j + 4
            cols.append(jnp.max(o1[:, lo:hi, :], axis=1, keepdims=True))
        pooled = jnp.concatenate(cols, axis=1).reshape(P * P, H2)

        pw_ref[...] = jnp.dot(pooled, wf_ref[...],
                              preferred_element_type=jnp.float32)
        hw_ref[...] = jnp.dot(h1_ref[...], wf_ref[:, :OUTP],
                              preferred_element_type=jnp.float32) + bf_ref[...]

    return pl.pallas_call(
        body,
        out_shape=(
            jax.ShapeDtypeStruct((P * P, GPAD), jnp.float32),
            jax.ShapeDtypeStruct((NP, OUTP), jnp.float32),
        ),
    )(bev3, h1, Wfp, bfp)


@functools.partial(
    pl.kernel,
    out_type=jax.ShapeDtypeStruct((NP, OUTP), jnp.float32),
    mesh=_MESH,
    scratch_types=[
        pltpu.VMEM((PB,), jnp.int32),
        pltpu.VMEM((PB, OUTP), jnp.float32),
        pltpu.VMEM((PB, GPAD), jnp.float32),
        pltpu.SemaphoreType.DMA,
    ],
)
def _final_sc(h1w, pooledw, pidx, out, pidx_v, hw_v, g_v, sem):
    cid = lax.axis_index("c")
    sid = lax.axis_index("s")
    wid = sid * NC + cid

    for u in range(PPW // PB):
        base = wid * PPW + u * PB
        pltpu.sync_copy(pidx.at[pl.ds(base, PB)], pidx_v)
        pltpu.sync_copy(h1w.at[pl.ds(base, PB)], hw_v)
        pltpu.async_copy(pooledw.at[pidx_v], g_v, sem).wait()

        def row(i, carry):
            for j in range(OUTP // 16):
                hw_v[i, pl.ds(j * 16, 16)] = (hw_v[i, pl.ds(j * 16, 16)]
                                              + g_v[i, pl.ds(j * 16, 16)])
            return carry
        lax.fori_loop(0, PB, row, 0)

        pltpu.sync_copy(hw_v, out.at[pl.ds(base, PB)])


def kernel(x, edge_index, coords, W0, W1, Wf, bf):
    src = edge_index[0]
    dst = edge_index[1]

    p = _segsum(x, src, dst)
    h0 = _tc1(p, W0)
    q = _segsum(h0, src, dst)
    h1 = _tc_h1(q, W1)

    xy = coords[:, :2] * (2.0 * 50.0) - 50.0
    px = jnp.clip(jnp.floor((xy[:, 0] + 50.0) / 0.5).astype(jnp.int32),
                  0, GRID - 1)
    py = jnp.clip((GRID - jnp.floor((xy[:, 1] + 50.0) / 0.5)).astype(jnp.int32)
                  - 1, 0, GRID - 1)
    bev = jnp.zeros((GRID, GRID, H2), jnp.float32).at[py, px].add(h1[:N])

    ppy = jnp.clip((py + 1) // 3, 0, P - 1)
    ppx = jnp.clip((px + 1) // 3, 0, P - 1)
    pidx = jnp.pad(ppy * P + ppx, (0, NP - N))

    Wfp = jnp.pad(Wf, ((0, 0), (0, GPAD - OUT)))
    bfp = jnp.pad(bf, (0, OUTP - OUT)).reshape(1, OUTP)
    pooledW, h1W = _tc2(bev, h1, Wfp, bfp)
    outp = _final_sc(h1W, pooledW, pidx)
    return outp[:N, :OUT]
